# baseline (speedup 1.0000x reference)
import jax
import jax.numpy as jnp
from jax.experimental import pallas as pl
from jax.experimental.pallas import tpu as pltpu

_N = 10000
_FEAT = 128
_NHID = 256

_TM = 200
_NT_M = _N // _TM


def _fused_body(adj_ref, x_ref, wt_ref, wb_ref, b_ref, w1_ref, b1_ref,
                alpha_ref, w2_ref, b2_ref, out_ref):
    m = pl.program_id(0)
    agg = jnp.dot(adj_ref[...], x_ref[...],
                  preferred_element_type=jnp.float32)
    xm = x_ref[pl.ds(m * _TM, _TM), :]
    h = jnp.dot(xm, wt_ref[...], preferred_element_type=jnp.float32)
    h += jnp.dot(agg, wb_ref[...], preferred_element_type=jnp.float32)
    h = jnp.maximum(h + b_ref[...], 0.0)
    h1 = jnp.dot(h, w1_ref[...], preferred_element_type=jnp.float32)
    h1 += b1_ref[...]
    h1 = jnp.where(h1 >= 0, h1, alpha_ref[...] * h1)
    pred = jnp.sum(h1 * w2_ref[...], axis=1) + b2_ref[0, 0]
    out_ref[0, 0, :] = pred


def kernel(x, adj, W, b, W1, b1, alpha, W2, b2):
    wt = W[:_FEAT]
    wb = W[_FEAT:]
    out = pl.pallas_call(
        _fused_body,
        grid=(_NT_M,),
        in_specs=[
            pl.BlockSpec((_TM, _N), lambda m: (m, 0)),
            pl.BlockSpec((_N, _FEAT), lambda m: (0, 0)),
            pl.BlockSpec((_FEAT, _NHID), lambda m: (0, 0)),
            pl.BlockSpec((_FEAT, _NHID), lambda m: (0, 0)),
            pl.BlockSpec((1, _NHID), lambda m: (0, 0)),
            pl.BlockSpec((_NHID, _NHID), lambda m: (0, 0)),
            pl.BlockSpec((1, _NHID), lambda m: (0, 0)),
            pl.BlockSpec((1, _NHID), lambda m: (0, 0)),
            pl.BlockSpec((1, _NHID), lambda m: (0, 0)),
            pl.BlockSpec((1, 1), lambda m: (0, 0)),
        ],
        out_specs=pl.BlockSpec((1, 1, _TM), lambda m: (m, 0, 0)),
        out_shape=jax.ShapeDtypeStruct((_NT_M, 1, _TM), jnp.float32),
        compiler_params=pltpu.CompilerParams(
            dimension_semantics=("arbitrary",),
        ),
    )(adj, x, wt, wb, b.reshape(1, _NHID), W1, b1.reshape(1, _NHID),
      alpha.reshape(1, _NHID), W2.reshape(1, _NHID), b2.reshape(1, 1))
    return out.reshape(-1)


# speedup vs baseline: 1.0002x; 1.0002x over previous
import jax
import jax.numpy as jnp
from jax.experimental import pallas as pl
from jax.experimental.pallas import tpu as pltpu

_N = 10000
_FEAT = 128
_NHID = 256

_TM = 200
_NT_M = _N // _TM


def _fused_body(adj_ref, x_ref, wt_ref, wb_ref, b_ref, w1_ref, b1_ref,
                alpha_ref, w2_ref, b2_ref, out_ref):
    m = pl.program_id(0)
    agg = jnp.dot(adj_ref[...], x_ref[...],
                  preferred_element_type=jnp.float32,
                  precision=jax.lax.Precision.DEFAULT)
    xm = x_ref[pl.ds(m * _TM, _TM), :]
    h = jnp.dot(xm, wt_ref[...], preferred_element_type=jnp.float32)
    h += jnp.dot(agg, wb_ref[...], preferred_element_type=jnp.float32)
    h = jnp.maximum(h + b_ref[...], 0.0)
    h1 = jnp.dot(h, w1_ref[...], preferred_element_type=jnp.float32)
    h1 += b1_ref[...]
    h1 = jnp.where(h1 >= 0, h1, alpha_ref[...] * h1)
    pred = jnp.sum(h1 * w2_ref[...], axis=1) + b2_ref[0, 0]
    out_ref[0, 0, :] = pred


def kernel(x, adj, W, b, W1, b1, alpha, W2, b2):
    wt = W[:_FEAT]
    wb = W[_FEAT:]
    out = pl.pallas_call(
        _fused_body,
        grid=(_NT_M,),
        in_specs=[
            pl.BlockSpec((_TM, _N), lambda m: (m, 0)),
            pl.BlockSpec((_N, _FEAT), lambda m: (0, 0)),
            pl.BlockSpec((_FEAT, _NHID), lambda m: (0, 0)),
            pl.BlockSpec((_FEAT, _NHID), lambda m: (0, 0)),
            pl.BlockSpec((1, _NHID), lambda m: (0, 0)),
            pl.BlockSpec((_NHID, _NHID), lambda m: (0, 0)),
            pl.BlockSpec((1, _NHID), lambda m: (0, 0)),
            pl.BlockSpec((1, _NHID), lambda m: (0, 0)),
            pl.BlockSpec((1, _NHID), lambda m: (0, 0)),
            pl.BlockSpec((1, 1), lambda m: (0, 0)),
        ],
        out_specs=pl.BlockSpec((1, 1, _TM), lambda m: (m, 0, 0)),
        out_shape=jax.ShapeDtypeStruct((_NT_M, 1, _TM), jnp.float32),
        compiler_params=pltpu.CompilerParams(
            dimension_semantics=("arbitrary",),
        ),
    )(adj, x, wt, wb, b.reshape(1, _NHID), W1, b1.reshape(1, _NHID),
      alpha.reshape(1, _NHID), W2.reshape(1, _NHID), b2.reshape(1, 1))
    return out.reshape(-1)


# speedup vs baseline: 1.0417x; 1.0415x over previous
import jax
import jax.numpy as jnp
from jax.experimental import pallas as pl
from jax.experimental.pallas import tpu as pltpu

_N = 10000
_FEAT = 128
_NHID = 256

_TMC = 80
_NC = _N // _TMC
_NBUF = 8
_TSUP = 2000
_CPS = _TSUP // _TMC
_NSUP = _N // _TSUP


def _outer(adj_hbm, x_ref, wt_ref, wb_ref, b_ref, w1_ref, b1_ref,
           alpha_ref, w2_ref, b2_ref, out_ref, agg_ref):

    def _chunk(adj_blk):
        i = pl.program_id(0)
        agg_ref[pl.ds(i * _TMC, _TMC), :] = jnp.dot(
            adj_blk[...], x_ref[...], preferred_element_type=jnp.float32)

        @pl.when(i % _CPS == _CPS - 1)
        def _epilogue():
            j = i // _CPS
            s = j * _TSUP
            xm = x_ref[pl.ds(s, _TSUP), :]
            agg = agg_ref[pl.ds(s, _TSUP), :]
            h = jnp.dot(xm, wt_ref[...], preferred_element_type=jnp.float32)
            h += jnp.dot(agg, wb_ref[...], preferred_element_type=jnp.float32)
            h = jnp.maximum(h + b_ref[...], 0.0)
            h1 = jnp.dot(h, w1_ref[...], preferred_element_type=jnp.float32)
            h1 += b1_ref[...]
            h1 = jnp.where(h1 >= 0, h1, alpha_ref[...] * h1)
            pred = jnp.sum(h1 * w2_ref[...], axis=1) + b2_ref[0, 0]
            out_ref[j, :] = pred

    pipe = pltpu.emit_pipeline(
        _chunk,
        grid=(_NC,),
        in_specs=[
            pl.BlockSpec((_TMC, _N), lambda i: (i, 0),
                         pipeline_mode=pl.Buffered(buffer_count=_NBUF)),
        ],
    )
    pipe(adj_hbm)


def kernel(x, adj, W, b, W1, b1, alpha, W2, b2):
    wt = W[:_FEAT]
    wb = W[_FEAT:]
    out = pl.pallas_call(
        _outer,
        grid=(1,),
        in_specs=[
            pl.BlockSpec(memory_space=pltpu.MemorySpace.HBM),
            pl.BlockSpec((_N, _FEAT), lambda i: (0, 0)),
            pl.BlockSpec((_FEAT, _NHID), lambda i: (0, 0)),
            pl.BlockSpec((_FEAT, _NHID), lambda i: (0, 0)),
            pl.BlockSpec((1, _NHID), lambda i: (0, 0)),
            pl.BlockSpec((_NHID, _NHID), lambda i: (0, 0)),
            pl.BlockSpec((1, _NHID), lambda i: (0, 0)),
            pl.BlockSpec((1, _NHID), lambda i: (0, 0)),
            pl.BlockSpec((1, _NHID), lambda i: (0, 0)),
            pl.BlockSpec((1, 1), lambda i: (0, 0)),
        ],
        out_specs=pl.BlockSpec((_NSUP, _TSUP), lambda i: (0, 0)),
        out_shape=jax.ShapeDtypeStruct((_NSUP, _TSUP), jnp.float32),
        scratch_shapes=[pltpu.VMEM((_N, _FEAT), jnp.float32)],
        compiler_params=pltpu.CompilerParams(
            dimension_semantics=("arbitrary",),
        ),
    )(adj, x, wt, wb, b.reshape(1, _NHID), W1, b1.reshape(1, _NHID),
      alpha.reshape(1, _NHID), W2.reshape(1, _NHID), b2.reshape(1, 1))
    return out.reshape(-1)
